# baseline (speedup 1.0000x reference)
import functools

import jax
import jax.numpy as jnp
from jax import lax
from jax.experimental import pallas as pl
from jax.experimental.pallas import tpu as pltpu
from jax.experimental.pallas import tpu_sc as plsc

EMBED_DIM = 32
BATCH = 16384
_NVOC = 1000000

_NC = 2
_NS = 16
_NW = _NC * _NS
_BPW = BATCH // _NW
_NBLK = _NVOC // 128
_TAIL0 = _NBLK * 128

_RING = 8


def _tt_body(uid_hbm, mid_hbm, ut_hbm, it_hbm, out_hbm,
             idx_u, idx_m, ring_u, ring_m, tail_u, tail_m, dots, tps,
             sem_u, sem_m):
    wid = lax.axis_index("s") * _NC + lax.axis_index("c")
    base = wid * _BPW

    for j in range(_BPW // 128):
        pltpu.sync_copy(uid_hbm.at[pl.ds(base + j * 128, 128)], idx_u.at[j])
        pltpu.sync_copy(mid_hbm.at[pl.ds(base + j * 128, 128)], idx_m.at[j])
    pltpu.sync_copy(ut_hbm.at[:, pl.ds(_TAIL0, 64)], tail_u)
    pltpu.sync_copy(it_hbm.at[:, pl.ds(_TAIL0, 64)], tail_m)

    lane = lax.iota(jnp.int32, 16)

    def fire(v, tab_hbm, ring, sem, k):
        sid = v[k]
        blk = jnp.minimum(lax.shift_right_logical(sid, 7), _NBLK - 1)
        off = pl.multiple_of(blk * 128, 128)
        return pltpu.async_copy(
            tab_hbm.at[:, pl.ds(off, 128)], ring.at[k % _RING], sem)

    def drain_quad():
        for k in range(4):
            pltpu.make_async_copy(
                ut_hbm.at[:, pl.ds(0, 128)], ring_u.at[k], sem_u).wait()
            pltpu.make_async_copy(
                it_hbm.at[:, pl.ds(0, 128)], ring_m.at[k], sem_m).wait()

    def extract(v, ring, tail, k):
        sid = v[k]
        tail_p = sid >= _TAIL0
        colf = sid & 127
        colt = jnp.minimum(sid - _TAIL0, 63)
        cf = jnp.full((16,), colf, jnp.int32)
        ct = jnp.full((16,), jnp.maximum(colt, 0), jnp.int32)
        r0 = plsc.load_gather(ring.at[k % _RING], [lane, cf])
        r1 = plsc.load_gather(ring.at[k % _RING], [lane + 16, cf])
        t0 = plsc.load_gather(tail, [lane, ct])
        t1 = plsc.load_gather(tail, [lane + 16, ct])
        return (jnp.where(tail_p, t0, r0), jnp.where(tail_p, t1, r1))

    def load_vregs(g):
        gc = jnp.minimum(g, _BPW // 16 - 1)
        gr = gc // 8
        go = (gc % 8) * 16
        return idx_u[gr, pl.ds(go, 16)], idx_m[gr, pl.ds(go, 16)]

    def fire_quad(v_u, v_m, q):
        for k in range(q * 4, q * 4 + 4):
            fire(v_u, ut_hbm, ring_u, sem_u, k)
            fire(v_m, it_hbm, ring_m, sem_m, k)

    def group(g, carry):
        v_u, v_m = load_vregs(g)
        v_un, v_mn = load_vregs(g + 1)

        def extract_quad(q):
            for k in range(q * 4, q * 4 + 4):
                u0, u1 = extract(v_u, ring_u, tail_u, k)
                m0, m1 = extract(v_m, ring_m, tail_m, k)
                s = u0 * m0 + u1 * m1
                plsc.store_scatter(tps, [lane * 17 + k], s)

        drain_quad()
        extract_quad(0)
        fire_quad(v_u, v_m, 2)
        drain_quad()
        extract_quad(1)
        fire_quad(v_u, v_m, 3)
        drain_quad()
        extract_quad(2)
        fire_quad(v_un, v_mn, 0)
        drain_quad()
        extract_quad(3)
        fire_quad(v_un, v_mn, 1)
        acc = tps[pl.ds(0, 16)]
        for i2 in range(1, 16):
            acc = acc + tps[pl.ds(i2 * 17, 16)]
        dots[pl.ds(g * 16, 16)] = 1.0 / (1.0 + jnp.exp(-acc))
        return carry

    v0_u, v0_m = load_vregs(0)
    fire_quad(v0_u, v0_m, 0)
    fire_quad(v0_u, v0_m, 1)
    lax.fori_loop(0, _BPW // 16, group, 0)
    drain_quad()
    drain_quad()
    pltpu.sync_copy(dots, out_hbm.at[pl.ds(base, _BPW)])


@jax.jit
def _two_tower(user_id, movie_id, user_table_t, item_table_t):
    mesh = plsc.VectorSubcoreMesh(core_axis_name="c", subcore_axis_name="s")
    return pl.kernel(
        _tt_body,
        out_type=jax.ShapeDtypeStruct((BATCH,), jnp.float32),
        mesh=mesh,
        compiler_params=pltpu.CompilerParams(
            needs_layout_passes=False, use_tc_tiling_on_sc=True),
        scratch_types=[
            pltpu.VMEM((_BPW // 128, 128), jnp.int32),
            pltpu.VMEM((_BPW // 128, 128), jnp.int32),
            pltpu.VMEM((_RING, EMBED_DIM, 128), jnp.float32),
            pltpu.VMEM((_RING, EMBED_DIM, 128), jnp.float32),
            pltpu.VMEM((EMBED_DIM, 64), jnp.float32),
            pltpu.VMEM((EMBED_DIM, 64), jnp.float32),
            pltpu.VMEM((_BPW,), jnp.float32),
            pltpu.VMEM((16 * 17,), jnp.float32),
            pltpu.SemaphoreType.DMA,
            pltpu.SemaphoreType.DMA,
        ],
    )(user_id, movie_id, user_table_t, item_table_t)


def kernel(user_id, movie_id, user_table, item_table):
    return _two_tower(user_id.astype(jnp.int32), movie_id.astype(jnp.int32),
                      user_table.T, item_table.T)


# speedup vs baseline: 1.1647x; 1.1647x over previous
import functools

import jax
import jax.numpy as jnp
from jax import lax
from jax.experimental import pallas as pl
from jax.experimental.pallas import tpu as pltpu
from jax.experimental.pallas import tpu_sc as plsc

EMBED_DIM = 32
BATCH = 16384
_NVOC = 1000000

_NC = 2
_NS = 16
_NW = _NC * _NS
_BPW = BATCH // _NW
_NBLK = _NVOC // 128
_TAIL0 = _NBLK * 128

_RING = 8


def _tt_body(uid_hbm, mid_hbm, ut_hbm, it_hbm, out_hbm,
             idx_u, idx_m, ring_u, ring_m, tail_u, tail_m, dots, tps,
             sem):
    wid = lax.axis_index("s") * _NC + lax.axis_index("c")
    base = wid * _BPW

    for j in range(_BPW // 128):
        pltpu.sync_copy(uid_hbm.at[pl.ds(base + j * 128, 128)], idx_u.at[j])
        pltpu.sync_copy(mid_hbm.at[pl.ds(base + j * 128, 128)], idx_m.at[j])
    pltpu.sync_copy(ut_hbm.at[:, pl.ds(_TAIL0, 64)], tail_u)
    pltpu.sync_copy(it_hbm.at[:, pl.ds(_TAIL0, 64)], tail_m)

    lane = lax.iota(jnp.int32, 16)

    def fire(v, tab_hbm, ring, k):
        sid = v[k]
        blk = jnp.minimum(lax.shift_right_logical(sid, 7), _NBLK - 1)
        off = pl.multiple_of(blk * 128, 128)
        return pltpu.async_copy(
            tab_hbm.at[:, pl.ds(off, 128)], ring.at[k % _RING], sem)

    def drain_quad():
        for k in range(4):
            pltpu.make_async_copy(
                ut_hbm.at[:, pl.ds(0, 128)], ring_u.at[k], sem).wait()
            pltpu.make_async_copy(
                it_hbm.at[:, pl.ds(0, 128)], ring_m.at[k], sem).wait()

    def extract(v, ring, tail, k):
        sid = v[k]
        tail_p = sid >= _TAIL0
        colf = sid & 127
        colt = jnp.minimum(sid - _TAIL0, 63)
        cf = jnp.full((16,), colf, jnp.int32)
        ct = jnp.full((16,), jnp.maximum(colt, 0), jnp.int32)
        r0 = plsc.load_gather(ring.at[k % _RING], [lane, cf])
        r1 = plsc.load_gather(ring.at[k % _RING], [lane + 16, cf])
        t0 = plsc.load_gather(tail, [lane, ct])
        t1 = plsc.load_gather(tail, [lane + 16, ct])
        return (jnp.where(tail_p, t0, r0), jnp.where(tail_p, t1, r1))

    def load_vregs(g):
        gc = jnp.minimum(g, _BPW // 16 - 1)
        gr = gc // 8
        go = (gc % 8) * 16
        return idx_u[gr, pl.ds(go, 16)], idx_m[gr, pl.ds(go, 16)]

    def fire_quad(v_u, v_m, q):
        for k in range(q * 4, q * 4 + 4):
            fire(v_u, ut_hbm, ring_u, k)
            fire(v_m, it_hbm, ring_m, k)

    def group(g, carry):
        v_u, v_m = load_vregs(g)
        v_un, v_mn = load_vregs(g + 1)

        def extract_quad(q):
            for k in range(q * 4, q * 4 + 4):
                u0, u1 = extract(v_u, ring_u, tail_u, k)
                m0, m1 = extract(v_m, ring_m, tail_m, k)
                s = u0 * m0 + u1 * m1
                plsc.store_scatter(tps, [lane * 17 + k], s)

        drain_quad()
        extract_quad(0)
        fire_quad(v_u, v_m, 2)
        drain_quad()
        extract_quad(1)
        fire_quad(v_u, v_m, 3)
        drain_quad()
        extract_quad(2)
        fire_quad(v_un, v_mn, 0)
        drain_quad()
        extract_quad(3)
        fire_quad(v_un, v_mn, 1)
        acc = tps[pl.ds(0, 16)]
        for i2 in range(1, 16):
            acc = acc + tps[pl.ds(i2 * 17, 16)]
        dots[pl.ds(g * 16, 16)] = 1.0 / (1.0 + jnp.exp(-acc))
        return carry

    v0_u, v0_m = load_vregs(0)
    fire_quad(v0_u, v0_m, 0)
    fire_quad(v0_u, v0_m, 1)
    lax.fori_loop(0, _BPW // 16, group, 0)
    drain_quad()
    drain_quad()
    pltpu.sync_copy(dots, out_hbm.at[pl.ds(base, _BPW)])


@jax.jit
def _two_tower(user_id, movie_id, user_table_t, item_table_t):
    mesh = plsc.VectorSubcoreMesh(core_axis_name="c", subcore_axis_name="s")
    return pl.kernel(
        _tt_body,
        out_type=jax.ShapeDtypeStruct((BATCH,), jnp.float32),
        mesh=mesh,
        compiler_params=pltpu.CompilerParams(
            needs_layout_passes=False, use_tc_tiling_on_sc=True),
        scratch_types=[
            pltpu.VMEM((_BPW // 128, 128), jnp.int32),
            pltpu.VMEM((_BPW // 128, 128), jnp.int32),
            pltpu.VMEM((_RING, EMBED_DIM, 128), jnp.float32),
            pltpu.VMEM((_RING, EMBED_DIM, 128), jnp.float32),
            pltpu.VMEM((EMBED_DIM, 64), jnp.float32),
            pltpu.VMEM((EMBED_DIM, 64), jnp.float32),
            pltpu.VMEM((_BPW,), jnp.float32),
            pltpu.VMEM((16 * 17,), jnp.float32),
            pltpu.SemaphoreType.DMA,
        ],
    )(user_id, movie_id, user_table_t, item_table_t)


def kernel(user_id, movie_id, user_table, item_table):
    return _two_tower(user_id.astype(jnp.int32), movie_id.astype(jnp.int32),
                      user_table.T, item_table.T)
